# Initial kernel scaffold; baseline (speedup 1.0000x reference)
import jax
import jax.numpy as jnp
from jax.experimental import pallas as pl


def kernel(x, emb):
    raise NotImplementedError("write your pallas kernel here")


# speedup vs baseline: 1.4573x; 1.4573x over previous
import functools

import jax
import jax.numpy as jnp
from jax.experimental import pallas as pl
from jax.experimental.pallas import tpu as pltpu

B = 4
S = 4096
D = 1024
S_BLK = 512


def _add_body(x_ref, emb_ref, out_ref):
    out_ref[...] = x_ref[...] + emb_ref[...][None]


@jax.jit
def kernel(x, emb):
    n_s = S // S_BLK
    grid = (B, n_s)
    out = pl.pallas_call(
        _add_body,
        grid=grid,
        in_specs=[
            pl.BlockSpec((1, S_BLK, D), lambda b, s: (b, s, 0)),
            pl.BlockSpec((S_BLK, D), lambda b, s: (s, 0)),
        ],
        out_specs=pl.BlockSpec((1, S_BLK, D), lambda b, s: (b, s, 0)),
        out_shape=jax.ShapeDtypeStruct((B, S, D), jnp.float32),
        compiler_params=pltpu.CompilerParams(
            dimension_semantics=("arbitrary", "arbitrary"),
        ),
    )(x, emb)
    return out


# speedup vs baseline: 1.6732x; 1.1482x over previous
import functools

import jax
import jax.numpy as jnp
from jax.experimental import pallas as pl
from jax.experimental.pallas import tpu as pltpu

B = 4
S = 4096
D = 1024
S_BLK = 512


def _add_body(x_ref, emb_ref, out_ref):
    out_ref[...] = x_ref[...] + emb_ref[...][None]


@jax.jit
def kernel(x, emb):
    n_s = S // S_BLK
    grid = (n_s, B)
    out = pl.pallas_call(
        _add_body,
        grid=grid,
        in_specs=[
            pl.BlockSpec((1, S_BLK, D), lambda s, b: (b, s, 0)),
            pl.BlockSpec((S_BLK, D), lambda s, b: (s, 0)),
        ],
        out_specs=pl.BlockSpec((1, S_BLK, D), lambda s, b: (b, s, 0)),
        out_shape=jax.ShapeDtypeStruct((B, S, D), jnp.float32),
        compiler_params=pltpu.CompilerParams(
            dimension_semantics=("arbitrary", "arbitrary"),
        ),
    )(x, emb)
    return out


# speedup vs baseline: 1.8517x; 1.1067x over previous
import functools

import jax
import jax.numpy as jnp
from jax.experimental import pallas as pl
from jax.experimental.pallas import tpu as pltpu

B = 4
S = 4096
D = 1024
S_BLK = 1024


def _add_body(x_ref, emb_ref, out_ref):
    out_ref[...] = x_ref[...] + emb_ref[...][None]


@jax.jit
def kernel(x, emb):
    n_s = S // S_BLK
    grid = (n_s, B)
    out = pl.pallas_call(
        _add_body,
        grid=grid,
        in_specs=[
            pl.BlockSpec((1, S_BLK, D), lambda s, b: (b, s, 0)),
            pl.BlockSpec((S_BLK, D), lambda s, b: (s, 0)),
        ],
        out_specs=pl.BlockSpec((1, S_BLK, D), lambda s, b: (b, s, 0)),
        out_shape=jax.ShapeDtypeStruct((B, S, D), jnp.float32),
        compiler_params=pltpu.CompilerParams(
            dimension_semantics=("arbitrary", "arbitrary"),
        ),
    )(x, emb)
    return out


# speedup vs baseline: 1.9645x; 1.0609x over previous
import functools

import jax
import jax.numpy as jnp
from jax.experimental import pallas as pl
from jax.experimental.pallas import tpu as pltpu

B = 4
S = 4096
D = 1024
S_BLK = 2048


def _add_body(x_ref, emb_ref, out_ref):
    out_ref[...] = x_ref[...] + emb_ref[...][None]


@jax.jit
def kernel(x, emb):
    n_s = S // S_BLK
    grid = (n_s, B)
    out = pl.pallas_call(
        _add_body,
        grid=grid,
        in_specs=[
            pl.BlockSpec((1, S_BLK, D), lambda s, b: (b, s, 0)),
            pl.BlockSpec((S_BLK, D), lambda s, b: (s, 0)),
        ],
        out_specs=pl.BlockSpec((1, S_BLK, D), lambda s, b: (b, s, 0)),
        out_shape=jax.ShapeDtypeStruct((B, S, D), jnp.float32),
        compiler_params=pltpu.CompilerParams(
            dimension_semantics=("arbitrary", "arbitrary"),
        ),
    )(x, emb)
    return out
